# Initial kernel scaffold; baseline (speedup 1.0000x reference)
import jax
import jax.numpy as jnp
from jax.experimental import pallas as pl


def kernel(atom_features, atom_edge_index, bond_features, radius_edge_features, atom_edge_sh, res_features, atom_res_batch, agg_edge_features, agg_edge_sh, rigid_quats, rigid_trans, res_edge_features, res_edge_index, res_mask, params):
    raise NotImplementedError("write your pallas kernel here")


# speedup vs baseline: 1.2599x; 1.2599x over previous
import functools
import numpy as np
import jax
import jax.numpy as jnp
from jax.experimental import pallas as pl

N_ATOM = 50000; E_BOND = 100000; E_RAD = 700000; E_ATOM = 800000
N_RES = 5000; E_RES = 150000
AS = 48; AV = 12; RS = 64; RV = 16; HE = 64; CS = 256; CZ = 128

EB_A = 2000
NB_B = 2000
EB_D = 2000

_F32 = jnp.float32
_dot = functools.partial(jnp.dot, preferred_element_type=jnp.float32)


def _full(shape):
    return pl.BlockSpec(shape, lambda i: (0,) * len(shape))


def _rows(blk, width):
    return pl.BlockSpec((blk, width), lambda i: (i, 0))


def _edge_msg_kernel(ef, s_src, v0, v1, v2, sh,
                     wb1, bb1, wb2, bb2, wr1, br1, wr2, br2, wvs, wsv,
                     out0, o1_0, o1_1, o1_2):
    is_bond = pl.program_id(0) < (E_BOND // EB_A)
    w1 = jnp.where(is_bond, wb1[...], wr1[...])
    b1 = jnp.where(is_bond, bb1[...], br1[...])
    w2 = jnp.where(is_bond, wb2[...], wr2[...])
    b2 = jnp.where(is_bond, bb2[...], br2[...])
    ew = _dot(jax.nn.relu(_dot(ef[...], w1) + b1), w2) + b2
    w0a = ew[:, :AS]; w0b = ew[:, AS:2 * AS]
    w1a = ew[:, 2 * AS:2 * AS + AV]; w1b = ew[:, 2 * AS + AV:]
    sh0 = sh[:, 0:1]
    s1_0 = sh[:, 1:2]; s1_1 = sh[:, 2:3]; s1_2 = sh[:, 3:4]
    va = v0[...]; vb = v1[...]; vc = v2[...]
    vdots = va * s1_0 + vb * s1_1 + vc * s1_2
    s = s_src[...]
    out0[...] = s * sh0 * w0a + _dot(vdots, wvs[...]) * w0b
    sv = _dot(s, wsv[...])
    o1_0[...] = sv * s1_0 * w1a + va * sh0 * w1b
    o1_1[...] = sv * s1_1 * w1a + vb * sh0 * w1b
    o1_2[...] = sv * s1_2 * w1a + vc * sh0 * w1b


def _agg_msg_kernel(af, agg, aef, sh, sel0, sel1, sel2,
                    w1, b1, w2, b2, wss, wvs, wsv, wvv,
                    ao, o0, o1_0, o1_1, o1_2):
    a = af[...] + agg[...] * np.float32(1.0 / np.sqrt(E_ATOM / N_ATOM))
    ao[...] = a
    s_a = a[:, :AS]
    vflat = a[:, AS:]
    va = _dot(vflat, sel0[...]); vb = _dot(vflat, sel1[...]); vc = _dot(vflat, sel2[...])
    ew = _dot(jax.nn.relu(_dot(aef[...], w1[...]) + b1[...]), w2[...]) + b2[...]
    sh0 = sh[:, 0:1]
    s1_0 = sh[:, 1:2]; s1_1 = sh[:, 2:3]; s1_2 = sh[:, 3:4]
    vdots = va * s1_0 + vb * s1_1 + vc * s1_2
    o0[...] = (_dot(s_a, wss[...]) * sh0 * ew[:, :RS]
               + _dot(vdots, wvs[...]) * ew[:, RS:2 * RS])
    ssv = _dot(s_a, wsv[...])
    ea = ew[:, 2 * RS:2 * RS + RV]; eb = ew[:, 2 * RS + RV:]
    vva = _dot(va, wvv[...]); vvb = _dot(vb, wvv[...]); vvc = _dot(vc, wvv[...])
    o1_0[...] = ssv * s1_0 * ea + vva * sh0 * eb
    o1_1[...] = ssv * s1_1 * ea + vvb * sh0 * eb
    o1_2[...] = ssv * s1_2 * ea + vvc * sh0 * eb


def _res_scalar_kernel(ru0, ru1_0, ru1_1, ru1_2, rflat, resf,
                       w1s, w1v0, w1v1, w1v2, b1, w2, b2, w3, b3, g, b,
                       sfeat):
    r = rflat[...]
    u0 = ru1_0[...]; u1 = ru1_1[...]; u2 = ru1_2[...]
    rot0 = r[:, 0:1] * u0 + r[:, 3:4] * u1 + r[:, 6:7] * u2
    rot1 = r[:, 1:2] * u0 + r[:, 4:5] * u1 + r[:, 7:8] * u2
    rot2 = r[:, 2:3] * u0 + r[:, 5:6] * u1 + r[:, 8:9] * u2
    h = (_dot(ru0[...], w1s[...]) + _dot(rot0, w1v0[...])
         + _dot(rot1, w1v1[...]) + _dot(rot2, w1v2[...]) + b1[...])
    h = jax.nn.relu(h)
    h = jax.nn.relu(_dot(h, w2[...]) + b2[...])
    h = _dot(h, w3[...]) + b3[...]
    x = resf[...] + h
    mu = jnp.mean(x, axis=-1, keepdims=True)
    var = jnp.mean((x - mu) * (x - mu), axis=-1, keepdims=True)
    sfeat[...] = (x - mu) * jax.lax.rsqrt(var + 1e-5) * g[...] + b[...]


def _ipmp_kernel(sr, sd, z, geo, wa, wb, wc, wd, b1, w2, b2, wz, bz,
                 m_out, z_out):
    h = (_dot(sr[...], wa[...]) + _dot(sd[...], wb[...])
         + _dot(z[...], wc[...]) + _dot(geo[...], wd[...]) + b1[...])
    m = _dot(jax.nn.relu(h), w2[...]) + b2[...]
    m_out[...] = m
    z_out[...] = z[...] + _dot(m, wz[...]) + bz[...]


def _node_update_kernel(sfeat, magg, mask, wu, bu, g, b, s_new):
    h = _dot(jax.nn.relu(magg[...]), wu[...]) + bu[...]
    x = sfeat[...] + h
    mu = jnp.mean(x, axis=-1, keepdims=True)
    var = jnp.mean((x - mu) * (x - mu), axis=-1, keepdims=True)
    y = (x - mu) * jax.lax.rsqrt(var + 1e-5) * g[...] + b[...]
    s_new[...] = y * mask[...]


def _quat_to_rot(q):
    q = q / (jnp.linalg.norm(q, axis=-1, keepdims=True) + 1e-8)
    w, x, y, z = q[..., 0], q[..., 1], q[..., 2], q[..., 3]
    r0 = jnp.stack([1 - 2*(y*y + z*z), 2*(x*y - w*z), 2*(x*z + w*y)], axis=-1)
    r1 = jnp.stack([2*(x*y + w*z), 1 - 2*(x*x + z*z), 2*(y*z - w*x)], axis=-1)
    r2 = jnp.stack([2*(x*z - w*y), 2*(y*z + w*x), 1 - 2*(x*x + y*y)], axis=-1)
    return jnp.stack([r0, r1, r2], axis=-2)


@jax.jit
def kernel(atom_features, atom_edge_index, bond_features, radius_edge_features,
           atom_edge_sh, res_features, atom_res_batch, agg_edge_features,
           agg_edge_sh, rigid_quats, rigid_trans, res_edge_features,
           res_edge_index, res_mask, params):
    p = params
    src = atom_edge_index[0]; dst = atom_edge_index[1]

    ef = jnp.concatenate([bond_features, radius_edge_features], axis=0)
    s_src = atom_features[:, :AS][src]
    v_src = atom_features[:, AS:][src].reshape(E_ATOM, AV, 3)
    b2d = lambda a: a.reshape(1, -1)
    grid_a = (E_ATOM // EB_A,)
    out0, o1_0, o1_1, o1_2 = pl.pallas_call(
        _edge_msg_kernel,
        grid=grid_a,
        in_specs=[_rows(EB_A, HE), _rows(EB_A, AS), _rows(EB_A, AV),
                  _rows(EB_A, AV), _rows(EB_A, AV), _rows(EB_A, 9),
                  _full((HE, HE)), _full((1, HE)), _full((HE, 2*AS+2*AV)),
                  _full((1, 2*AS+2*AV)),
                  _full((HE, HE)), _full((1, HE)), _full((HE, 2*AS+2*AV)),
                  _full((1, 2*AS+2*AV)),
                  _full((AV, AS)), _full((AS, AV))],
        out_specs=[_rows(EB_A, AS), _rows(EB_A, AV), _rows(EB_A, AV),
                   _rows(EB_A, AV)],
        out_shape=[jax.ShapeDtypeStruct((E_ATOM, AS), _F32),
                   jax.ShapeDtypeStruct((E_ATOM, AV), _F32),
                   jax.ShapeDtypeStruct((E_ATOM, AV), _F32),
                   jax.ShapeDtypeStruct((E_ATOM, AV), _F32)],
    )(ef, s_src, v_src[:, :, 0], v_src[:, :, 1], v_src[:, :, 2], atom_edge_sh,
      p['ac_Wb1'], b2d(p['ac_bb1']), p['ac_Wb2'], b2d(p['ac_bb2']),
      p['ac_Wr1'], b2d(p['ac_br1']), p['ac_Wr2'], b2d(p['ac_br2']),
      p['ac_Wvs'], p['ac_Wsv'])
    out1 = jnp.stack([o1_0, o1_1, o1_2], axis=-1).reshape(E_ATOM, 3 * AV)
    msg = jnp.concatenate([out0, out1], axis=-1)
    agg = jax.ops.segment_sum(msg, dst, num_segments=N_ATOM)

    sel = np.zeros((3, 3 * AV, AV), np.float32)
    for k in range(3):
        for c in range(AV):
            sel[k, 3 * c + k, c] = 1.0
    sel = jnp.asarray(sel)
    grid_b = (N_ATOM // NB_B,)
    atom_out, o0, q1_0, q1_1, q1_2 = pl.pallas_call(
        _agg_msg_kernel,
        grid=grid_b,
        in_specs=[_rows(NB_B, AS + 3*AV), _rows(NB_B, AS + 3*AV),
                  _rows(NB_B, HE), _rows(NB_B, 9),
                  _full((3*AV, AV)), _full((3*AV, AV)), _full((3*AV, AV)),
                  _full((HE, HE)), _full((1, HE)),
                  _full((HE, 2*RS+2*RV)), _full((1, 2*RS+2*RV)),
                  _full((AS, RS)), _full((AV, RS)), _full((AS, RV)),
                  _full((AV, RV))],
        out_specs=[_rows(NB_B, AS + 3*AV), _rows(NB_B, RS), _rows(NB_B, RV),
                   _rows(NB_B, RV), _rows(NB_B, RV)],
        out_shape=[jax.ShapeDtypeStruct((N_ATOM, AS + 3*AV), _F32),
                   jax.ShapeDtypeStruct((N_ATOM, RS), _F32),
                   jax.ShapeDtypeStruct((N_ATOM, RV), _F32),
                   jax.ShapeDtypeStruct((N_ATOM, RV), _F32),
                   jax.ShapeDtypeStruct((N_ATOM, RV), _F32)],
    )(atom_features, agg, agg_edge_features, agg_edge_sh,
      sel[0], sel[1], sel[2],
      p['gc_W1'], b2d(p['gc_b1']), p['gc_W2'], b2d(p['gc_b2']),
      p['gc_Wss'], p['gc_Wvs'], p['gc_Wsv'], p['gc_Wvv'])
    o1 = jnp.stack([q1_0, q1_1, q1_2], axis=-1).reshape(N_ATOM, 3 * RV)
    res_msg = jnp.concatenate([o0, o1], axis=-1)
    res_update = jax.ops.segment_sum(res_msg, atom_res_batch,
                                     num_segments=N_RES)
    res_update = res_update * np.float32(1.0 / np.sqrt(N_ATOM / N_RES))

    R = _quat_to_rot(rigid_quats)
    rflat = R.reshape(N_RES, 9)
    ru0 = res_update[:, :RS]
    ru1 = res_update[:, RS:].reshape(N_RES, RV, 3)
    w1 = p['i2s_W1']
    w1v = w1[RS:].reshape(RV, 3, CS)
    sfeat = pl.pallas_call(
        _res_scalar_kernel,
        grid=(1,),
        in_specs=[_rows(N_RES, RS), _rows(N_RES, RV), _rows(N_RES, RV),
                  _rows(N_RES, RV), _rows(N_RES, 9), _rows(N_RES, CS),
                  _full((RS, CS)), _full((RV, CS)), _full((RV, CS)),
                  _full((RV, CS)), _full((1, CS)),
                  _full((CS, CS)), _full((1, CS)),
                  _full((CS, CS)), _full((1, CS)),
                  _full((1, CS)), _full((1, CS))],
        out_specs=_rows(N_RES, CS),
        out_shape=jax.ShapeDtypeStruct((N_RES, CS), _F32),
    )(ru0, ru1[:, :, 0], ru1[:, :, 1], ru1[:, :, 2], rflat, res_features,
      w1[:RS], w1v[:, 0, :], w1v[:, 1, :], w1v[:, 2, :], b2d(p['i2s_b1']),
      p['i2s_W2'], b2d(p['i2s_b2']), p['i2s_W3'], b2d(p['i2s_b3']),
      b2d(p['ln_g']), b2d(p['ln_b']))

    rs = res_edge_index[0]; rd = res_edge_index[1]
    rel = rigid_trans[rs] - rigid_trans[rd]
    Rinv = jnp.swapaxes(R, -1, -2)
    local = jnp.einsum('eij,ej->ei', Rinv[rd], rel)
    dist = jnp.linalg.norm(rel, axis=-1, keepdims=True)
    geo = jnp.concatenate([local, dist], axis=-1)
    wm1 = p['ip_Wm1']
    m, z_new = pl.pallas_call(
        _ipmp_kernel,
        grid=(E_RES // EB_D,),
        in_specs=[_rows(EB_D, CS), _rows(EB_D, CS), _rows(EB_D, CZ),
                  _rows(EB_D, 4),
                  _full((CS, CS)), _full((CS, CS)), _full((CZ, CS)),
                  _full((4, CS)), _full((1, CS)),
                  _full((CS, CS)), _full((1, CS)),
                  _full((CS, CZ)), _full((1, CZ))],
        out_specs=[_rows(EB_D, CS), _rows(EB_D, CZ)],
        out_shape=[jax.ShapeDtypeStruct((E_RES, CS), _F32),
                   jax.ShapeDtypeStruct((E_RES, CZ), _F32)],
    )(sfeat[rs], sfeat[rd], res_edge_features, geo,
      wm1[:CS], wm1[CS:2*CS], wm1[2*CS:2*CS+CZ], wm1[2*CS+CZ:],
      b2d(p['ip_bm1']), p['ip_Wm2'], b2d(p['ip_bm2']),
      p['ip_Wz'], b2d(p['ip_bz']))
    magg = jax.ops.segment_sum(m, rd, num_segments=N_RES)
    magg = magg * np.float32(1.0 / np.sqrt(E_RES / N_RES))

    s_new = pl.pallas_call(
        _node_update_kernel,
        grid=(1,),
        in_specs=[_rows(N_RES, CS), _rows(N_RES, CS), _rows(N_RES, 1),
                  _full((CS, CS)), _full((1, CS)),
                  _full((1, CS)), _full((1, CS))],
        out_specs=_rows(N_RES, CS),
        out_shape=jax.ShapeDtypeStruct((N_RES, CS), _F32),
    )(sfeat, magg, res_mask.reshape(N_RES, 1),
      p['ip_Wu'], b2d(p['ip_bu']), b2d(p['ip_g']), b2d(p['ip_b']))

    return atom_out, s_new, z_new
